# Initial kernel scaffold; baseline (speedup 1.0000x reference)
import jax
import jax.numpy as jnp
from jax.experimental import pallas as pl


def kernel(A_weight, A_bias, V_weight, V_bias, out_weight, out_bias, in_weight, in_bias, Aww, Awb, Abw, Abb, AencW, Aencb, Vww, Vwb, Vbw, Vbb, VencW, Vencb, Oww, Owb, Obw, Obb, OencW, Oencb, Iww, Iwb, Ibw, Ibb, IencW, Iencb, SW, Sb, Wih, Whh, bih, bhh, FW, Fb, gnn_layers, A_wp, A_bp, V_wp, V_bp, out_wp, out_bp, in_wp, in_bp):
    raise NotImplementedError("write your pallas kernel here")


# speedup vs baseline: 1.6374x; 1.6374x over previous
import jax
import jax.numpy as jnp
from jax import lax
from jax.experimental import pallas as pl
from jax.experimental.pallas import tpu as pltpu

_INTERPRET = False


def _msum(rows, lens_col):
    ci = lax.broadcasted_iota(jnp.int32, rows.shape, 1)
    return jnp.sum(jnp.where(ci < lens_col, rows, 0.0), axis=1, keepdims=True)


def _emb(sw, lw, sb, lb, ww, wb, bw, bb, eww, ewb, eb):
    aw = jax.nn.sigmoid(sw * ww + lw * wb)
    ab = jax.nn.sigmoid(sb * bw + lb * bb)
    return jax.nn.relu(
        jnp.dot(aw, eww, preferred_element_type=jnp.float32)
        + jnp.dot(ab, ewb, preferred_element_type=jnp.float32) + eb)


def _tc_body(
    aw_ref, ab_ref, vw_ref, vb_ref,
    ow_ref, ob_ref, iw_ref, ib_ref,
    law_ref, lab_ref, lvw_ref, lvb_ref,
    lawg_ref, labg_ref, lvwg_ref, lvbg_ref,
    low_ref, lob_ref, liw_ref, lib_ref,
    glen_ref,
    aww_ref, awb_ref, abw_ref, abb_ref, aeww_ref, aewb_ref, aeb_ref,
    vww_ref, vwb_ref, vbw_ref, vbb_ref, veww_ref, vewb_ref, veb_ref,
    oww_ref, owb_ref, obw_ref, obb_ref, oeww_ref, oewb_ref, oeb_ref,
    iww_ref, iwb_ref, ibw_ref, ibb_ref, ieww_ref, iewb_ref, ieb_ref,
    sw_ref, sb_ref, wiha_ref, wihv_ref, whh_ref, bias_ref,
    fwh_ref, fwo_ref, fb_ref,
    out_ref, embA_s, embV_s,
):
    B, G = 16, 32
    R = B * G
    sA = jnp.concatenate([
        _msum(aw_ref[...], law_ref[...]),
        _msum(ab_ref[...], lab_ref[...]),
        _msum(vw_ref[...], lvw_ref[...]),
        _msum(vb_ref[...], lvb_ref[...]),
    ], axis=1)

    r_i = lax.broadcasted_iota(jnp.int32, (R, R), 0)
    c_i = lax.broadcasted_iota(jnp.int32, (R, R), 1)
    perm = jnp.where(c_i == (r_i % B) * G + r_i // B, 1.0, 0.0)
    sAg = jnp.dot(perm.astype(jnp.float32), sA,
                  preferred_element_type=jnp.float32)

    embA_s[...] = _emb(sAg[:, 0:1], lawg_ref[...], sAg[:, 1:2], labg_ref[...],
                       aww_ref[...], awb_ref[...], abw_ref[...], abb_ref[...],
                       aeww_ref[...], aewb_ref[...], aeb_ref[...])
    embV_s[...] = _emb(sAg[:, 2:3], lvwg_ref[...], sAg[:, 3:4], lvbg_ref[...],
                       vww_ref[...], vwb_ref[...], vbw_ref[...], vbb_ref[...],
                       veww_ref[...], vewb_ref[...], veb_ref[...])

    out_emb = _emb(_msum(ow_ref[...], low_ref[...]),
                   low_ref[...].astype(jnp.float32),
                   _msum(ob_ref[...], lob_ref[...]),
                   lob_ref[...].astype(jnp.float32),
                   oww_ref[...], owb_ref[...], obw_ref[...], obb_ref[...],
                   oeww_ref[...], oewb_ref[...], oeb_ref[...])
    in_emb = _emb(_msum(iw_ref[...], liw_ref[...]),
                  liw_ref[...].astype(jnp.float32),
                  _msum(ib_ref[...], lib_ref[...]),
                  lib_ref[...].astype(jnp.float32),
                  iww_ref[...], iwb_ref[...], ibw_ref[...], ibb_ref[...],
                  ieww_ref[...], iewb_ref[...], ieb_ref[...])

    h0 = jnp.dot(in_emb, sw_ref[...], preferred_element_type=jnp.float32) \
        + sb_ref[...]
    glen = glen_ref[...]
    wiha = wiha_ref[...]
    wihv = wihv_ref[...]
    whh = whh_ref[...]
    bias = bias_ref[...]

    def step(t, h):
        xA = embA_s[pl.ds(t * B, B), :]
        xV = embV_s[pl.ds(t * B, B), :]
        pre = (jnp.dot(xA, wiha, preferred_element_type=jnp.float32)
               + jnp.dot(xV, wihv, preferred_element_type=jnp.float32)
               + jnp.dot(h, whh, preferred_element_type=jnp.float32) + bias)
        return jnp.where(glen > t, jnp.tanh(pre), h)

    h = lax.fori_loop(0, G, step, h0)
    out_ref[...] = (jnp.dot(h, fwh_ref[...], preferred_element_type=jnp.float32)
                    + jnp.dot(out_emb, fwo_ref[...],
                              preferred_element_type=jnp.float32)
                    + fb_ref[...])


def kernel(A_weight, A_bias, V_weight, V_bias, out_weight, out_bias,
           in_weight, in_bias, Aww, Awb, Abw, Abb, AencW, Aencb,
           Vww, Vwb, Vbw, Vbb, VencW, Vencb, Oww, Owb, Obw, Obb, OencW, Oencb,
           Iww, Iwb, Ibw, Ibb, IencW, Iencb, SW, Sb, Wih, Whh, bih, bhh,
           FW, Fb, gnn_layers, A_wp, A_bp, V_wp, V_bp, out_wp, out_bp,
           in_wp, in_bp):
    B, G, PW = A_weight.shape
    PB = A_bias.shape[2]
    TT = Aww.shape[0]
    H = AencW.shape[1]
    RH = Whh.shape[0]

    lengths = gnn_layers.astype(jnp.int32)
    si = jnp.argsort(-lengths, stable=True)
    ui = jnp.argsort(si, stable=True)
    ls = lengths[si]
    bs = jnp.sum(ls[None, :] > jnp.arange(G, dtype=jnp.int32)[:, None], axis=1)
    csum_b = jnp.concatenate([jnp.zeros((1,), bs.dtype), jnp.cumsum(bs)[:-1]])
    pid = jnp.arange(B, dtype=bs.dtype)[:, None] + csum_b[None, :]
    cg = jnp.cumsum(lengths)
    k = jnp.searchsorted(cg, pid.reshape(-1), side='right')
    k = jnp.clip(k, 0, B - 1).reshape(B, G)
    k_nat = k[ui]

    def lens2(wp):
        m = wp.astype(jnp.int32)[k_nat]
        return m.reshape(B * G, 1), m.T.reshape(B * G, 1).astype(jnp.float32)

    law, lawg = lens2(A_wp)
    lab, labg = lens2(A_bp)
    lvw, lvwg = lens2(V_wp)
    lvb, lvbg = lens2(V_bp)

    r2 = lambda x: x.reshape(1, -1)
    f32 = jnp.float32
    out = pl.pallas_call(
        _tc_body,
        out_shape=jax.ShapeDtypeStruct((B, 256), f32),
        scratch_shapes=[pltpu.VMEM((B * G, H), f32),
                        pltpu.VMEM((B * G, H), f32)],
        interpret=_INTERPRET,
    )(
        A_weight.reshape(B * G, PW), A_bias.reshape(B * G, PB),
        V_weight.reshape(B * G, PW), V_bias.reshape(B * G, PB),
        out_weight, out_bias, in_weight, in_bias,
        law, lab, lvw, lvb, lawg, labg, lvwg, lvbg,
        out_wp.astype(jnp.int32).reshape(B, 1),
        out_bp.astype(jnp.int32).reshape(B, 1),
        in_wp.astype(jnp.int32).reshape(B, 1),
        in_bp.astype(jnp.int32).reshape(B, 1),
        lengths.reshape(B, 1),
        r2(Aww), r2(Awb), r2(Abw), r2(Abb), AencW[:TT], AencW[TT:], r2(Aencb),
        r2(Vww), r2(Vwb), r2(Vbw), r2(Vbb), VencW[:TT], VencW[TT:], r2(Vencb),
        r2(Oww), r2(Owb), r2(Obw), r2(Obb), OencW[:TT], OencW[TT:], r2(Oencb),
        r2(Iww), r2(Iwb), r2(Ibw), r2(Ibb), IencW[:TT], IencW[TT:], r2(Iencb),
        SW, r2(Sb), Wih[:H], Wih[H:], Whh, r2(bih + bhh),
        FW[:RH], FW[RH:], r2(Fb),
    )
    return out


# speedup vs baseline: 11.2571x; 6.8749x over previous
import jax
import jax.numpy as jnp
from jax import lax
from jax.experimental import pallas as pl
from jax.experimental.pallas import tpu as pltpu

_INTERPRET = False
_B, _G = 16, 32
_TT, _T, _H, _RH = 64, 8, 128, 256


def _msum(rows, lens_col):
    ci = lax.broadcasted_iota(jnp.int32, rows.shape, 1).astype(jnp.float32)
    return jnp.sum(jnp.where(ci < lens_col, rows, 0.0), axis=1, keepdims=True)


def _emb(sw, lw, sb, lb, ww, wb, bw, bb, eww, ewb, eb):
    aw = jax.nn.sigmoid(sw * ww + lw * wb)
    ab = jax.nn.sigmoid(sb * bw + lb * bb)
    return jax.nn.relu(
        jnp.dot(aw, eww, preferred_element_type=jnp.float32)
        + jnp.dot(ab, ewb, preferred_element_type=jnp.float32) + eb)


def _dot(a, b):
    return jnp.dot(a, b, preferred_element_type=jnp.float32)


def _tc_body(
    aw_ref, ab_ref, vw_ref, vb_ref,
    ow_ref, ob_ref, iw_ref, ib_ref,
    glen_c_ref, glen_r_ref,
    wp4_ref,
    low_ref, lob_ref, liw_ref, lib_ref,
    aww_ref, awb_ref, abw_ref, abb_ref, aencw_ref, aencb_ref,
    vww_ref, vwb_ref, vbw_ref, vbb_ref, vencw_ref, vencb_ref,
    oww_ref, owb_ref, obw_ref, obb_ref, oencw_ref, oencb_ref,
    iww_ref, iwb_ref, ibw_ref, ibb_ref, iencw_ref, iencb_ref,
    sw_ref, sb_ref, wih_ref, whh_ref, bih_ref, bhh_ref,
    fw_ref, fb_ref,
    out_ref, lens_g_s, embA_s, embV_s, x_s,
):
    B, G = _B, _G
    R = B * G
    f32 = jnp.float32

    lf_c = glen_c_ref[...].astype(f32)
    lf_r = glen_r_ref[...].astype(f32)
    t_row = lax.broadcasted_iota(jnp.int32, (1, G), 1).astype(f32)
    bs_row = jnp.sum(
        jnp.where(lf_c > t_row, 1.0, 0.0), axis=0, keepdims=True)
    r32 = lax.broadcasted_iota(jnp.int32, (G, G), 0)
    c32 = lax.broadcasted_iota(jnp.int32, (G, G), 1)
    lt_s32 = jnp.where(r32 < c32, 1.0, 0.0).astype(f32)
    csum_b = _dot(bs_row, lt_s32)
    r16 = lax.broadcasted_iota(jnp.int32, (B, B), 0)
    c16 = lax.broadcasted_iota(jnp.int32, (B, B), 1)
    le_16 = jnp.where(r16 <= c16, 1.0, 0.0).astype(f32)
    cg_row = _dot(lf_r, le_16)
    pid = lax.broadcasted_iota(jnp.int32, (B, G), 0).astype(f32) + csum_b
    bi_c = lax.broadcasted_iota(jnp.int32, (B, 1), 0)
    bi_r = lax.broadcasted_iota(jnp.int32, (1, B), 1)
    beats = (lf_r > lf_c) | ((lf_r == lf_c) & (bi_r < bi_c))
    ui_col = jnp.sum(jnp.where(beats, 1.0, 0.0), axis=1, keepdims=True)
    i_row16 = lax.broadcasted_iota(jnp.int32, (1, B), 1).astype(f32)
    perm16 = jnp.where(ui_col == i_row16, 1.0, 0.0).astype(f32)
    wp4 = wp4_ref[...].astype(f32)
    for t in range(G):
        pid_t = pid[:, t:t + 1]
        k_t = jnp.sum(jnp.where(cg_row <= pid_t, 1.0, 0.0),
                      axis=1, keepdims=True)
        k_t = jnp.minimum(k_t, float(B - 1))
        kn_t = _dot(perm16, k_t)
        oh = jnp.where(kn_t == i_row16, 1.0, 0.0).astype(f32)
        lens_g_s[t * B:(t + 1) * B, :] = _dot(oh, wp4)
    lens_g = lens_g_s[...]
    rr = lax.broadcasted_iota(jnp.int32, (R, R), 0)
    cc = lax.broadcasted_iota(jnp.int32, (R, R), 1)
    perm_g2b = jnp.where(cc == (rr % G) * B + rr // G, 1.0, 0.0).astype(f32)
    perm_b2g = jnp.where(cc == (rr % B) * G + rr // B, 1.0, 0.0).astype(f32)
    lens_b = _dot(perm_g2b, lens_g)

    sA = jnp.concatenate([
        _msum(aw_ref[...], lens_b[:, 0:1]),
        _msum(ab_ref[...], lens_b[:, 1:2]),
        _msum(vw_ref[...], lens_b[:, 2:3]),
        _msum(vb_ref[...], lens_b[:, 3:4]),
    ], axis=1)
    sAg = _dot(perm_b2g, sA)

    embA_s[...] = _emb(sAg[:, 0:1], lens_g[:, 0:1], sAg[:, 1:2], lens_g[:, 1:2],
                       aww_ref[...], awb_ref[...], abw_ref[...], abb_ref[...],
                       aencw_ref[:_TT], aencw_ref[_TT:], aencb_ref[...])
    embV_s[...] = _emb(sAg[:, 2:3], lens_g[:, 2:3], sAg[:, 3:4], lens_g[:, 3:4],
                       vww_ref[...], vwb_ref[...], vbw_ref[...], vbb_ref[...],
                       vencw_ref[:_TT], vencw_ref[_TT:], vencb_ref[...])

    low = low_ref[...].astype(f32)
    lob = lob_ref[...].astype(f32)
    liw = liw_ref[...].astype(f32)
    lib = lib_ref[...].astype(f32)
    out_emb = _emb(_msum(ow_ref[...], low), low, _msum(ob_ref[...], lob), lob,
                   oww_ref[...], owb_ref[...], obw_ref[...], obb_ref[...],
                   oencw_ref[:_TT], oencw_ref[_TT:], oencb_ref[...])
    in_emb = _emb(_msum(iw_ref[...], liw), liw, _msum(ib_ref[...], lib), lib,
                  iww_ref[...], iwb_ref[...], ibw_ref[...], ibb_ref[...],
                  iencw_ref[:_TT], iencw_ref[_TT:], iencb_ref[...])

    x_s[...] = (_dot(embA_s[...], wih_ref[:_H])
                + _dot(embV_s[...], wih_ref[_H:])
                + bih_ref[...] + bhh_ref[...])
    h = _dot(in_emb, sw_ref[...]) + sb_ref[...]
    glen = glen_c_ref[...]
    whh = whh_ref[...]
    for t in range(G):
        pre = x_s[t * B:(t + 1) * B, :] + _dot(h, whh)
        h = jnp.where(glen > t, jnp.tanh(pre), h)
    out_ref[...] = (_dot(h, fw_ref[:_RH]) + _dot(out_emb, fw_ref[_RH:])
                    + fb_ref[...])


def kernel(A_weight, A_bias, V_weight, V_bias, out_weight, out_bias,
           in_weight, in_bias, Aww, Awb, Abw, Abb, AencW, Aencb,
           Vww, Vwb, Vbw, Vbb, VencW, Vencb, Oww, Owb, Obw, Obb, OencW, Oencb,
           Iww, Iwb, Ibw, Ibb, IencW, Iencb, SW, Sb, Wih, Whh, bih, bhh,
           FW, Fb, gnn_layers, A_wp, A_bp, V_wp, V_bp, out_wp, out_bp,
           in_wp, in_bp):
    B, G, PW = A_weight.shape
    PB = A_bias.shape[2]
    H = AencW.shape[1]
    f32 = jnp.float32
    i32 = jnp.int32
    r2 = lambda x: x.reshape(1, -1)
    c2 = lambda x: x.astype(i32).reshape(B, 1)
    glen = gnn_layers.astype(i32)
    wp4 = jnp.stack([A_wp, A_bp, V_wp, V_bp], axis=1).astype(i32)

    out = pl.pallas_call(
        _tc_body,
        out_shape=jax.ShapeDtypeStruct((B, 256), f32),
        scratch_shapes=[pltpu.VMEM((B * G, 4), f32),
                        pltpu.VMEM((B * G, H), f32),
                        pltpu.VMEM((B * G, H), f32),
                        pltpu.VMEM((B * G, 256), f32)],
        interpret=_INTERPRET,
    )(
        A_weight.reshape(B * G, PW), A_bias.reshape(B * G, PB),
        V_weight.reshape(B * G, PW), V_bias.reshape(B * G, PB),
        out_weight, out_bias, in_weight, in_bias,
        glen.reshape(B, 1), glen.reshape(1, B), wp4,
        c2(out_wp), c2(out_bp), c2(in_wp), c2(in_bp),
        r2(Aww), r2(Awb), r2(Abw), r2(Abb), AencW, r2(Aencb),
        r2(Vww), r2(Vwb), r2(Vbw), r2(Vbb), VencW, r2(Vencb),
        r2(Oww), r2(Owb), r2(Obw), r2(Obb), OencW, r2(Oencb),
        r2(Iww), r2(Iwb), r2(Ibw), r2(Ibb), IencW, r2(Iencb),
        SW, r2(Sb), Wih, Whh, r2(bih), r2(bhh),
        FW, r2(Fb),
    )
    return out
